# baseline (speedup 1.0000x reference)
import functools

import jax
import jax.numpy as jnp
from jax import lax
from jax.experimental import pallas as pl
from jax.experimental.pallas import tpu as pltpu
from jax.experimental.pallas import tpu_sc as plsc

DIM = 128
VOCAB = 100
CHUNK = 128
NBUF = 5
UNROLL = 16
LANES = 16


def _project_body(emb_ref, w_ref, b_ref, out_ref):
    out_ref[...] = lax.dot_general(
        emb_ref[...], w_ref[...],
        dimension_numbers=(((1,), (1,)), ((), ())),
        preferred_element_type=jnp.float32,
    ) + b_ref[...]


def _project_table(emb_table, W, b):
    return pl.pallas_call(
        _project_body,
        out_shape=jax.ShapeDtypeStruct((VOCAB, DIM), jnp.float32),
    )(emb_table, W, b.reshape(1, DIM))


def _make_sc_gather(n_rows):
    info = plsc.get_sparse_core_info()
    nc, ns = info.num_cores, info.num_subcores
    nw = nc * ns
    assert n_rows % (nw * CHUNK * NBUF) == 0
    per_w = n_rows // nw
    n_chunks = per_w // CHUNK
    n_iters = n_chunks // NBUF
    mesh = plsc.VectorSubcoreMesh(core_axis_name="c", subcore_axis_name="s")

    scratch = (
        [pltpu.VMEM_SHARED((VOCAB, DIM), jnp.float32)]
        + [pltpu.VMEM((CHUNK,), jnp.int32) for _ in range(NBUF)]
        + [pltpu.VMEM((CHUNK, DIM), jnp.float32) for _ in range(NBUF)]
        + [pltpu.SemaphoreType.DMA] * (3 * NBUF + 1)
    )

    @functools.partial(
        pl.kernel,
        mesh=mesh,
        out_type=jax.ShapeDtypeStruct((n_rows, DIM), jnp.float32),
        scratch_types=scratch,
    )
    def sc_gather(table_hbm, idx_hbm, out_hbm, *bufs):
        table_sh = bufs[0]
        idx_v = bufs[1:1 + NBUF]
        rows_v = bufs[1 + NBUF:1 + 2 * NBUF]
        idx_sem = bufs[1 + 2 * NBUF:1 + 3 * NBUF]
        gat_sem = bufs[1 + 3 * NBUF:1 + 4 * NBUF]
        out_sem = bufs[1 + 4 * NBUF:1 + 5 * NBUF]
        tab_sem = bufs[1 + 5 * NBUF]
        sid = lax.axis_index("s")
        wid = sid * nc + lax.axis_index("c")
        base = wid * per_w

        def idx_copy(g, b):
            off = base + g * CHUNK
            return pltpu.make_async_copy(
                idx_hbm.at[pl.ds(off, CHUNK)], idx_v[b], idx_sem[b])

        def gat_copy(b):
            return pltpu.make_async_copy(
                table_sh.at[idx_v[b]], rows_v[b], gat_sem[b])

        def out_copy(g, b):
            off = base + g * CHUNK
            return pltpu.make_async_copy(
                rows_v[b], out_hbm.at[pl.ds(off, CHUNK)], out_sem[b])

        for b in range(NBUF):
            idx_copy(b, b).start()

        @pl.when(sid == 0)
        def _():
            pltpu.make_async_copy(table_hbm, table_sh, tab_sem).start()
            pltpu.make_async_copy(table_hbm, table_sh, tab_sem).wait()

        plsc.subcore_barrier()

        def body(j, carry):
            g0 = j * NBUF
            for b in range(NBUF):
                idx_copy(g0 + b, b).wait()

                @pl.when(j > 0)
                def _(b=b):
                    out_copy(g0 + b - NBUF, b).wait()

                gat_copy(b).start()
            for b in range(NBUF):
                gat_copy(b).wait()
                out_copy(g0 + b, b).start()

                @pl.when(j < n_iters - 1)
                def _(b=b):
                    idx_copy(g0 + b + NBUF, b).start()
            return carry

        lax.fori_loop(0, n_iters, body, 0)
        for b in range(NBUF):
            out_copy(n_chunks - NBUF + b, b).wait()

    return sc_gather


def kernel(x, emb_table, W, b):
    batch, hist = x.shape
    table_proj = _project_table(emb_table, W, b)
    flat_idx = x.reshape(-1)
    gather = _make_sc_gather(batch * hist)
    out = gather(table_proj, flat_idx)
    return out.reshape(batch, hist, DIM)


# speedup vs baseline: 1.0073x; 1.0073x over previous
import functools

import jax
import jax.numpy as jnp
from jax import lax
from jax.experimental import pallas as pl
from jax.experimental.pallas import tpu as pltpu
from jax.experimental.pallas import tpu_sc as plsc

DIM = 128
VOCAB = 100
CHUNK = 128
NBUF = 4
UNROLL = 16
LANES = 16


def _project_body(emb_ref, w_ref, b_ref, out_ref):
    out_ref[...] = lax.dot_general(
        emb_ref[...], w_ref[...],
        dimension_numbers=(((1,), (1,)), ((), ())),
        preferred_element_type=jnp.float32,
    ) + b_ref[...]


def _project_table(emb_table, W, b):
    return pl.pallas_call(
        _project_body,
        out_shape=jax.ShapeDtypeStruct((VOCAB, DIM), jnp.float32),
    )(emb_table, W, b.reshape(1, DIM))


def _make_sc_gather(n_rows):
    info = plsc.get_sparse_core_info()
    nc, ns = info.num_cores, info.num_subcores
    nw = nc * ns
    assert n_rows % (nw * CHUNK * NBUF) == 0
    per_w = n_rows // nw
    n_chunks = per_w // CHUNK
    n_iters = n_chunks // NBUF
    mesh = plsc.VectorSubcoreMesh(core_axis_name="c", subcore_axis_name="s")

    assert n_iters % 2 == 0
    scratch = (
        [pltpu.VMEM_SHARED((VOCAB, DIM), jnp.float32)]
        + [pltpu.VMEM((NBUF, CHUNK), jnp.int32) for _ in range(2)]
        + [pltpu.VMEM((CHUNK, DIM), jnp.float32) for _ in range(NBUF)]
        + [pltpu.SemaphoreType.DMA] * (2 * NBUF + 3)
    )

    @functools.partial(
        pl.kernel,
        mesh=mesh,
        out_type=jax.ShapeDtypeStruct((n_rows, DIM), jnp.float32),
        scratch_types=scratch,
    )
    def sc_gather(table_hbm, idx_hbm, out_hbm, *bufs):
        table_sh = bufs[0]
        idx_blk = bufs[1:3]
        rows_v = bufs[3:3 + NBUF]
        gat_sem = bufs[3 + NBUF:3 + 2 * NBUF]
        out_sem = bufs[3 + 2 * NBUF:3 + 3 * NBUF]
        blk_sem = bufs[3 + 3 * NBUF:5 + 3 * NBUF]
        tab_sem = bufs[5 + 3 * NBUF]
        sid = lax.axis_index("s")
        wid = sid * nc + lax.axis_index("c")
        base = wid * per_w
        cbase = wid * n_chunks

        def blk_copy(j, p):
            return pltpu.make_async_copy(
                idx_hbm.at[pl.ds(cbase + j * NBUF, NBUF)],
                idx_blk[p], blk_sem[p])

        def gat_copy(b, p):
            return pltpu.make_async_copy(
                table_sh.at[idx_blk[p].at[b]], rows_v[b], gat_sem[b])

        def out_copy(g, b):
            off = base + g * CHUNK
            return pltpu.make_async_copy(
                rows_v[b], out_hbm.at[pl.ds(off, CHUNK)], out_sem[b])

        blk_copy(0, 0).start()
        blk_copy(1, 1).start()

        @pl.when(sid == 0)
        def _():
            pltpu.make_async_copy(table_hbm, table_sh, tab_sem).start()
            pltpu.make_async_copy(table_hbm, table_sh, tab_sem).wait()

        plsc.subcore_barrier()

        def half(j, p):
            g0 = j * NBUF
            blk_copy(j, p).wait()
            for b in range(NBUF):
                @pl.when(j > 0)
                def _(b=b):
                    out_copy(g0 + b - NBUF, b).wait()

                gat_copy(b, p).start()
            for b in range(NBUF):
                gat_copy(b, p).wait()
                out_copy(g0 + b, b).start()

            @pl.when(j < n_iters - 2)
            def _():
                blk_copy(j + 2, p).start()

        def body(jj, carry):
            half(2 * jj, 0)
            half(2 * jj + 1, 1)
            return carry

        lax.fori_loop(0, n_iters // 2, body, 0)
        for b in range(NBUF):
            out_copy(n_chunks - NBUF + b, b).wait()

    return sc_gather


def kernel(x, emb_table, W, b):
    batch, hist = x.shape
    table_proj = _project_table(emb_table, W, b)
    idx2d = x.reshape(-1, CHUNK)
    gather = _make_sc_gather(batch * hist)
    out = gather(table_proj, idx2d)
    return out.reshape(batch, hist, DIM)


# speedup vs baseline: 1.0086x; 1.0013x over previous
import functools

import jax
import jax.numpy as jnp
from jax import lax
from jax.experimental import pallas as pl
from jax.experimental.pallas import tpu as pltpu
from jax.experimental.pallas import tpu_sc as plsc

DIM = 128
VOCAB = 100
CHUNK = 128
NBUF = 4


def _project_body(emb_ref, w_ref, b_ref, out_ref):
    out_ref[...] = lax.dot_general(
        emb_ref[...], w_ref[...],
        dimension_numbers=(((1,), (1,)), ((), ())),
        preferred_element_type=jnp.float32,
    ) + b_ref[...]


def _project_table(emb_table, W, b):
    return pl.pallas_call(
        _project_body,
        out_shape=jax.ShapeDtypeStruct((VOCAB, DIM), jnp.float32),
    )(emb_table, W, b.reshape(1, DIM))


def _make_sc_gather(n_rows):
    info = plsc.get_sparse_core_info()
    nc, ns = info.num_cores, info.num_subcores
    nw = nc * ns
    assert n_rows % (nw * CHUNK * NBUF) == 0
    per_w = n_rows // nw
    n_chunks = per_w // CHUNK
    n_iters = n_chunks // NBUF
    mesh = plsc.VectorSubcoreMesh(core_axis_name="c", subcore_axis_name="s")

    assert n_iters % 2 == 0
    scratch = (
        [pltpu.VMEM_SHARED((VOCAB, DIM), jnp.float32)]
        + [pltpu.VMEM((NBUF, CHUNK), jnp.int32) for _ in range(2)]
        + [pltpu.VMEM((CHUNK, DIM), jnp.float32) for _ in range(NBUF)]
        + [pltpu.SemaphoreType.DMA] * (2 * NBUF + 3)
    )

    @functools.partial(
        pl.kernel,
        mesh=mesh,
        out_type=jax.ShapeDtypeStruct((n_rows, DIM), jnp.float32),
        scratch_types=scratch,
    )
    def sc_gather(table_hbm, idx_hbm, out_hbm, *bufs):
        table_sh = bufs[0]
        idx_blk = bufs[1:3]
        rows_v = bufs[3:3 + NBUF]
        gat_sem = bufs[3 + NBUF:3 + 2 * NBUF]
        out_sem = bufs[3 + 2 * NBUF:3 + 3 * NBUF]
        blk_sem = bufs[3 + 3 * NBUF:5 + 3 * NBUF]
        tab_sem = bufs[5 + 3 * NBUF]
        sid = lax.axis_index("s")
        wid = sid * nc + lax.axis_index("c")
        base = wid * per_w
        cbase = wid * n_chunks

        def blk_copy(j, p):
            return pltpu.make_async_copy(
                idx_hbm.at[pl.ds(cbase + j * NBUF, NBUF)],
                idx_blk[p], blk_sem[p])

        def gat_copy(b, p):
            return pltpu.make_async_copy(
                table_sh.at[idx_blk[p].at[b]], rows_v[b], gat_sem[b])

        def out_copy(g, b):
            off = base + g * CHUNK
            return pltpu.make_async_copy(
                rows_v[b], out_hbm.at[pl.ds(off, CHUNK)], out_sem[b])

        blk_copy(0, 0).start()
        blk_copy(1, 1).start()

        @pl.when(sid == 0)
        def _():
            pltpu.make_async_copy(table_hbm, table_sh, tab_sem).start()
            pltpu.make_async_copy(table_hbm, table_sh, tab_sem).wait()

        plsc.subcore_barrier()

        def half(j, p):
            g0 = j * NBUF
            blk_copy(j, p).wait()
            for b in range(NBUF):
                @pl.when(j > 0)
                def _(b=b):
                    out_copy(g0 + b - NBUF, b).wait()

                gat_copy(b, p).start()
            for b in range(NBUF):
                gat_copy(b, p).wait()
                out_copy(g0 + b, b).start()

            @pl.when(j < n_iters - 2)
            def _():
                blk_copy(j + 2, p).start()

        def body(jj, carry):
            half(2 * jj, 0)
            half(2 * jj + 1, 1)
            return carry

        lax.fori_loop(0, n_iters // 2, body, 0)
        for b in range(NBUF):
            out_copy(n_chunks - NBUF + b, b).wait()

    return sc_gather


def kernel(x, emb_table, W, b):
    batch, hist = x.shape
    table_proj = _project_table(emb_table, W, b)
    idx2d = x.reshape(-1, CHUNK)
    gather = _make_sc_gather(batch * hist)
    out = gather(table_proj, idx2d)
    return out.reshape(batch, hist, DIM)
